# baseline (speedup 1.0000x reference)
import jax
import jax.numpy as jnp
from jax.experimental import pallas as pl

GROUPS = 1024
GSIZE = 64
N = 16384
RROWS = 128


def _fps_body(x_ref, c_ref):
    x0 = x_ref[0, 0]
    x1 = x_ref[0, 1]
    x2 = x_ref[0, 2]
    rows = jax.lax.broadcasted_iota(jnp.int32, (RROWS, 128), 0)
    cols = jax.lax.broadcasted_iota(jnp.int32, (RROWS, 128), 1)
    flat = rows * 128 + cols
    crows = jax.lax.broadcasted_iota(jnp.int32, (8, 128), 0)
    ccols = jax.lax.broadcasted_iota(jnp.int32, (8, 128), 1)

    def step(s, carry):
        dist, f, a0, a1, a2 = carry
        m = flat == f
        c0 = jnp.sum(jnp.where(m, x0, 0.0), keepdims=True)
        c1 = jnp.sum(jnp.where(m, x1, 0.0), keepdims=True)
        c2 = jnp.sum(jnp.where(m, x2, 0.0), keepdims=True)
        sm = (crows == s // 128) & (ccols == s % 128)
        a0 = jnp.where(sm, c0, a0)
        a1 = jnp.where(sm, c1, a1)
        a2 = jnp.where(sm, c2, a2)
        d0 = x0 - c0
        d1 = x1 - c1
        d2 = x2 - c2
        d = (d0 * d0 + d1 * d1) + d2 * d2
        dist = jnp.minimum(dist, d)
        v = jnp.max(dist, keepdims=True)
        f = jnp.min(jnp.where(dist == v, flat, N), keepdims=True)
        return dist, f, a0, a1, a2

    init = (
        jnp.full((RROWS, 128), 1e10, dtype=jnp.float32),
        jnp.zeros((1, 1), dtype=jnp.int32),
        jnp.zeros((8, 128), dtype=jnp.float32),
        jnp.zeros((8, 128), dtype=jnp.float32),
        jnp.zeros((8, 128), dtype=jnp.float32),
    )
    _, _, a0, a1, a2 = jax.lax.fori_loop(0, GROUPS, step, init)
    c_ref[0, 0] = a0
    c_ref[0, 1] = a1
    c_ref[0, 2] = a2


def _fps_centers(xyz):
    B = xyz.shape[0]
    xg = jnp.transpose(xyz, (0, 2, 1)).reshape(B, 3, RROWS, 128)
    cacc = pl.pallas_call(
        _fps_body,
        grid=(B,),
        in_specs=[pl.BlockSpec((1, 3, RROWS, 128), lambda b: (b, 0, 0, 0))],
        out_specs=pl.BlockSpec((1, 3, 8, 128), lambda b: (b, 0, 0, 0)),
        out_shape=jax.ShapeDtypeStruct((B, 3, 8, 128), jnp.float32),
    )(xg)
    return jnp.transpose(cacc.reshape(B, 3, GROUPS), (0, 2, 1))


CTILE = 256


def _knn_body(cx_ref, cy_ref, cz_ref, x0_ref, x1_ref, x2_ref, idx_ref):
    cx = cx_ref[0]
    cy = cy_ref[0]
    cz = cz_ref[0]
    x0 = x0_ref[0]
    x1 = x1_ref[0]
    x2 = x2_ref[0]
    d = (cx * cx + cy * cy + cz * cz) + (x0 * x0 + x1 * x1 + x2 * x2) \
        - 2.0 * (cx * x0 + cy * x1 + cz * x2)

    def step(t, carry):
        d, acc = carry
        cols = jax.lax.broadcasted_iota(jnp.int32, (CTILE, N), 1)
        v = jnp.min(d, axis=1, keepdims=True)
        w = jnp.min(jnp.where(d == v, cols, N), axis=1, keepdims=True)
        d = jnp.where(cols == w, 3.0e38, d)
        t64 = jax.lax.broadcasted_iota(jnp.int32, (CTILE, GSIZE), 1)
        acc = jnp.where(t64 == t, w, acc)
        return d, acc

    init = (d, jnp.zeros((CTILE, GSIZE), dtype=jnp.int32))
    _, acc = jax.lax.fori_loop(0, GSIZE, step, init)
    idx_ref[0] = acc


def _knn_idx(center, xyz):
    B = xyz.shape[0]
    c = jnp.transpose(center, (0, 2, 1))[..., None]
    x = jnp.transpose(xyz, (0, 2, 1))[:, :, None, :]
    grid = (B, GROUPS // CTILE)
    cspec = [pl.BlockSpec((1, CTILE, 1), lambda b, g, k=k: (b, g, 0)) for k in range(3)]
    xspec = [pl.BlockSpec((1, 1, N), lambda b, g: (b, 0, 0))] * 3
    return pl.pallas_call(
        _knn_body,
        grid=grid,
        in_specs=cspec + xspec,
        out_specs=pl.BlockSpec((1, CTILE, GSIZE), lambda b, g: (b, g, 0)),
        out_shape=jax.ShapeDtypeStruct((B, GROUPS, GSIZE), jnp.int32),
    )(c[:, 0], c[:, 1], c[:, 2], x[:, 0], x[:, 1], x[:, 2])


def kernel(xyz):
    B, n, _ = xyz.shape
    center = _fps_centers(xyz)
    idx = _knn_idx(center, xyz)
    idx_base = (jnp.arange(B, dtype=idx.dtype) * n)[:, None, None]
    idx_full = (idx + idx_base).reshape(-1)
    flat = xyz.reshape(B * n, 3)
    neighborhood = flat[idx_full].reshape(B, GROUPS, GSIZE, 3)
    neighborhood = neighborhood - center[:, :, None, :]
    return (neighborhood, center)
